# Initial kernel scaffold; baseline (speedup 1.0000x reference)
import jax
import jax.numpy as jnp
from jax.experimental import pallas as pl


def kernel(X, edge_index_0, edge_value_0, edge_index_1, edge_value_1, edge_index_2, edge_value_2, target_x, target, conv_weight, Ws0, Ws1, linear1_W, linear1_b, lin_W, lin_b):
    raise NotImplementedError("write your pallas kernel here")


# speedup vs baseline: 1.0002x; 1.0002x over previous
import jax
import jax.numpy as jnp
from jax.experimental import pallas as pl

BETA = 0.5
C = 2
N = 50000
D_OUT = 64


def kernel(X, edge_index_0, edge_value_0, edge_index_1, edge_value_1,
           edge_index_2, edge_value_2, target_x, target,
           conv_weight, Ws0, Ws1, linear1_W, linear1_b, lin_W, lin_b):
    filt = jax.nn.softmax(conv_weight, axis=1)
    edge_index = jnp.concatenate([edge_index_0, edge_index_1, edge_index_2], axis=1)
    row = edge_index[0]
    col = edge_index[1]
    X_list = [X @ Ws0, X @ Ws1]
    H_list = [X @ Ws0, X @ Ws1]
    H_new = []
    for i in range(C):
        vals = jnp.concatenate([
            edge_value_0 * filt[i, 0],
            edge_value_1 * filt[i, 1],
            edge_value_2 * filt[i, 2],
        ])
        msg = vals[:, None] * H_list[i][col]
        Hi = jnp.zeros((N, D_OUT), dtype=jnp.float32).at[row].add(msg)
        H_new.append(Hi)
    parts = [jax.nn.relu(BETA * X_list[i] + (1.0 - BETA) * H_new[i]) for i in range(C)]
    H_cat = jnp.concatenate(parts, axis=1)
    H_out = jax.nn.relu(H_cat @ linear1_W + linear1_b)
    y = H_out[target_x] @ lin_W + lin_b
    logp = jax.nn.log_softmax(y, axis=1)
    loss = -jnp.mean(jnp.take_along_axis(logp, target[:, None], axis=1))
    return (loss, y, filt)


# speedup vs baseline: 20.2220x; 20.2189x over previous
import functools

import jax
import jax.numpy as jnp
from jax import lax
from jax.experimental import pallas as pl
from jax.experimental.pallas import tpu as pltpu
from jax.experimental.pallas import tpu_sc as plsc

N = 50000
E = 200000
W_IN = 128
T = 5000
NUM_CLASS = 16

NW = 32
REG = 1664
NPAD = NW * REG
TPAD = 5120
TLEN = TPAD // NW
SPAD = 5120
ECAP = 3328
CCAP = ECAP + 32
NCHUNK = ECAP // 16

_mesh = plsc.VectorSubcoreMesh(core_axis_name="c", subcore_axis_name="s")
_sc_params = pltpu.CompilerParams(needs_layout_passes=False)


def _wid():
    return lax.axis_index("s") * 2 + lax.axis_index("c")


def _iota16():
    return lax.iota(jnp.int32, 16)


@functools.partial(
    pl.kernel,
    out_type=jax.ShapeDtypeStruct((NPAD,), jnp.int32),
    mesh=_mesh,
    compiler_params=_sc_params,
    scratch_types=[
        pltpu.VMEM((REG,), jnp.int32),
        pltpu.VMEM((TPAD,), jnp.int32),
    ],
)
def _k1_slot_map(tgt_hbm, out_hbm, buf, tgtv):
    w = _wid()
    n0 = w * REG

    def fill(i, _):
        buf[pl.ds(i * 16, 16)] = jnp.full((16,), -1, jnp.int32)
        return 0

    lax.fori_loop(0, REG // 16, fill, 0)
    pltpu.sync_copy(tgt_hbm, tgtv)

    def scat(i, _):
        tv = tgtv[pl.ds(i * 16, 16)]
        tid = _iota16() + i * 16
        m = (tid < T) & (tv >= n0) & (tv < n0 + REG)
        plsc.store_scatter(buf, [tv - n0], tid, mask=m)
        return 0

    lax.fori_loop(0, TPAD // 16, scat, 0)
    pltpu.sync_copy(buf, out_hbm.at[pl.ds(n0, REG)])


HALF = SPAD // 2


@functools.partial(
    pl.kernel,
    out_type=jax.ShapeDtypeStruct((2, SPAD, W_IN), jnp.float32),
    mesh=_mesh,
    compiler_params=_sc_params,
    scratch_types=[
        pltpu.VMEM((NPAD,), jnp.int32),
        pltpu.VMEM((ECAP,), jnp.int32),
        pltpu.VMEM((ECAP,), jnp.int32),
        pltpu.VMEM((ECAP,), jnp.float32),
        pltpu.VMEM((CCAP,), jnp.int32),
        pltpu.VMEM((CCAP,), jnp.int32),
        pltpu.VMEM((CCAP,), jnp.float32),
        pltpu.VMEM((6, 16), jnp.float32),
        pltpu.VMEM((16, W_IN), jnp.float32),
        pltpu.VMEM((16, W_IN), jnp.float32),
        pltpu.VMEM((16, W_IN), jnp.float32),
        pltpu.VMEM((16,), jnp.int32),
        pltpu.VMEM((32, W_IN), jnp.float32),
        pltpu.VMEM_SHARED((HALF, W_IN), jnp.float32),
        pltpu.VMEM_SHARED((HALF, W_IN), jnp.float32),
    ],
)
def _k2_edge_pass(smap_hbm, x_hbm, filt_hbm, dst0, src0, ev0, dst1, src1,
                  ev1, dst2, src2, ev2, accs_out,
                  smap_v, dst_v, src_v, val_v, slot_c, src_c, val_c, filt_v,
                  rows_v, w0rows, w1rows, widx_v, zbuf, acc0, acc1):
    c = lax.axis_index("c")
    s = lax.axis_index("s")
    lo = c * HALF
    z16 = jnp.zeros((16,), jnp.float32)
    zi16 = jnp.zeros((16,), jnp.int32)

    def zinit(i, _):
        slot_c[pl.ds(i * 16, 16)] = zi16
        src_c[pl.ds(i * 16, 16)] = zi16
        val_c[pl.ds(i * 16, 16)] = z16
        return 0

    lax.fori_loop(0, CCAP // 16, zinit, 0)
    for r in range(32):
        for q in range(W_IN // 16):
            zbuf[r, pl.ds(q * 16, 16)] = z16
    rows_per_sub = HALF // 16
    for acc in (acc0, acc1):
        def zacc(i, _, acc=acc):
            pltpu.sync_copy(zbuf, acc.at[pl.ds(s * rows_per_sub + i * 32, 32)])
            return 0
        lax.fori_loop(0, rows_per_sub // 32, zacc, 0)
    pltpu.sync_copy(smap_hbm, smap_v)
    pltpu.sync_copy(filt_hbm, filt_v)
    plsc.subcore_barrier()

    npieces = 64
    epw = E // npieces
    for k, (dst, src, ev) in enumerate(((dst0, src0, ev0),
                                        (dst1, src1, ev1),
                                        (dst2, src2, ev2))):
        f0k = filt_v[k, :]
        f1k = filt_v[3 + k, :]
        for h in range(4):
            p = s * 4 + h
            rw = (p * epw) // 128 * 128
            rw1 = jnp.where(p == npieces - 1, E, ((p + 1) * epw) // 128 * 128)
            lenw = rw1 - rw
            rwl = jnp.minimum(rw, E - ECAP)
            d = rw - rwl
            pltpu.sync_copy(dst.at[pl.ds(rwl, ECAP)], dst_v)
            pltpu.sync_copy(src.at[pl.ds(rwl, ECAP)], src_v)
            pltpu.sync_copy(ev.at[pl.ds(rwl, ECAP)], val_v)

            def compact(i, off):
                p0 = i * 16
                dvec = dst_v[pl.ds(p0, 16)]
                svec = src_v[pl.ds(p0, 16)]
                vvec = val_v[pl.ds(p0, 16)]
                pos = _iota16() + p0
                inb = (pos >= d) & (pos < d + lenw)
                slots = plsc.load_gather(smap_v, [dvec]) - lo
                m = (slots >= 0) & (slots < HALF) & inb
                cnt = jnp.max(plsc.all_reduce_population_count(m))
                plsc.store_compressed(slot_c.at[pl.ds(off, 16)], slots, mask=m)
                plsc.store_compressed(src_c.at[pl.ds(off, 16)], svec, mask=m)
                plsc.store_compressed(val_c.at[pl.ds(off, 16)], vvec, mask=m)
                return off + cnt

            nv = lax.fori_loop(0, NCHUNK, compact, jnp.int32(0))
            val_c[pl.ds(nv, 16)] = z16
            val_c[pl.ds(nv + 16, 16)] = z16
            nb = (nv + 15) // 16

            def batch(b, _):
                o = b * 16
                widx_v[...] = slot_c[pl.ds(o, 16)]
                svec = src_c[pl.ds(o, 16)]
                pltpu.sync_copy(x_hbm.at[svec], rows_v)
                for j in range(16):
                    wsp = plsc.load_gather(
                        val_c, [jnp.full((16,), o + j, jnp.int32)])
                    w0 = wsp * f0k
                    w1 = wsp * f1k
                    for q in range(W_IN // 16):
                        r = rows_v[j, pl.ds(q * 16, 16)]
                        w0rows[j, pl.ds(q * 16, 16)] = r * w0
                        w1rows[j, pl.ds(q * 16, 16)] = r * w1
                pltpu.sync_copy(w0rows, acc0.at[widx_v], add=True)
                pltpu.sync_copy(w1rows, acc1.at[widx_v], add=True)
                return 0

            lax.fori_loop(0, nb, batch, 0)

    plsc.subcore_barrier()
    for i, acc in enumerate((acc0, acc1)):
        pltpu.sync_copy(
            acc.at[pl.ds(s * rows_per_sub, rows_per_sub)],
            accs_out.at[i].at[pl.ds(lo + s * rows_per_sub, rows_per_sub)],
        )


@functools.partial(
    pl.kernel,
    out_type=(
        jax.ShapeDtypeStruct((2, TPAD, W_IN), jnp.float32),
        jax.ShapeDtypeStruct((TPAD, W_IN), jnp.float32),
    ),
    mesh=_mesh,
    compiler_params=_sc_params,
    scratch_types=[
        pltpu.VMEM((NPAD,), jnp.int32),
        pltpu.VMEM((TPAD,), jnp.int32),
        pltpu.VMEM((TLEN,), jnp.int32),
        pltpu.VMEM((TLEN,), jnp.int32),
        pltpu.VMEM((TLEN, W_IN), jnp.float32),
    ],
)
def _k3_gather(accs_hbm, smap_hbm, tgt_hbm, x_hbm, g_out, xt_out,
               smap_v, tgt_full, tgt_v, slot_v, pbuf):
    w = _wid()
    t0 = w * TLEN
    pltpu.sync_copy(smap_hbm, smap_v)
    pltpu.sync_copy(tgt_hbm, tgt_full)

    def slots(i, _):
        tv = tgt_full[pl.ds(t0 + i * 16, 16)]
        tgt_v[pl.ds(i * 16, 16)] = tv
        sv = plsc.load_gather(smap_v, [tv])
        slot_v[pl.ds(i * 16, 16)] = jnp.maximum(sv, 0)
        return 0

    lax.fori_loop(0, TLEN // 16, slots, 0)
    for p in range(2):
        pltpu.sync_copy(accs_hbm.at[p].at[slot_v], pbuf)
        pltpu.sync_copy(pbuf, g_out.at[p].at[pl.ds(t0, TLEN)])
    pltpu.sync_copy(x_hbm.at[tgt_v], pbuf)
    pltpu.sync_copy(pbuf, xt_out.at[pl.ds(t0, TLEN)])


def _k4_body(g_ref, xt_ref, cw_ref, ws0_ref, ws1_ref, w1_ref, b1_ref,
             lw_ref, lb_ref, tgt_ref, y_ref, loss_ref, filt_ref):
    cw = cw_ref[...]
    mx = jnp.max(cw, axis=1, keepdims=True)
    ex = jnp.exp(cw - mx)
    filt = ex / jnp.sum(ex, axis=1, keepdims=True)
    filt_ref[...] = filt

    xt = xt_ref[...]
    m_list = [0.5 * xt + 0.5 * g_ref[0], 0.5 * xt + 0.5 * g_ref[1]]
    p0 = jnp.maximum(jnp.dot(m_list[0], ws0_ref[...],
                             preferred_element_type=jnp.float32), 0.0)
    p1 = jnp.maximum(jnp.dot(m_list[1], ws1_ref[...],
                             preferred_element_type=jnp.float32), 0.0)
    hcat = jnp.concatenate([p0, p1], axis=1)
    ho = jnp.maximum(jnp.dot(hcat, w1_ref[...],
                             preferred_element_type=jnp.float32)
                     + b1_ref[...], 0.0)
    yf = jnp.dot(ho, lw_ref[...], preferred_element_type=jnp.float32) + lb_ref[...]
    y = yf[:T]
    y_ref[...] = y
    ymax = jnp.max(y, axis=1, keepdims=True)
    lse = jnp.log(jnp.sum(jnp.exp(y - ymax), axis=1, keepdims=True)) + ymax
    logp = y - lse
    tgt = tgt_ref[...]
    oh = lax.broadcasted_iota(jnp.int32, (T, NUM_CLASS), 1) == tgt
    pick = jnp.sum(jnp.where(oh, logp, 0.0), axis=1)
    loss_ref[...] = (-jnp.sum(pick) / T).reshape(1, 1)


_k4_head = pl.pallas_call(
    _k4_body,
    out_shape=(
        jax.ShapeDtypeStruct((T, NUM_CLASS), jnp.float32),
        jax.ShapeDtypeStruct((1, 1), jnp.float32),
        jax.ShapeDtypeStruct((2, 3), jnp.float32),
    ),
)


def kernel(X, edge_index_0, edge_value_0, edge_index_1, edge_value_1,
           edge_index_2, edge_value_2, target_x, target,
           conv_weight, Ws0, Ws1, linear1_W, linear1_b, lin_W, lin_b):
    tgt_pad = jnp.concatenate(
        [target_x, jnp.zeros((TPAD - T,), jnp.int32)])
    filt_w = jax.nn.softmax(conv_weight, axis=1)
    filt_pad = jnp.repeat(filt_w.reshape(6, 1), 16, axis=1)
    slot_map = _k1_slot_map(tgt_pad)
    accs = _k2_edge_pass(
        slot_map, X, filt_pad,
        edge_index_0[0], edge_index_0[1], edge_value_0,
        edge_index_1[0], edge_index_1[1], edge_value_1,
        edge_index_2[0], edge_index_2[1], edge_value_2)
    g, xt = _k3_gather(accs, slot_map, tgt_pad, X)
    y, loss, filt = _k4_head(g, xt, conv_weight, Ws0, Ws1,
                             linear1_W, linear1_b.reshape(1, -1),
                             lin_W, lin_b.reshape(1, -1),
                             target.reshape(T, 1))
    return (loss[0, 0], y, filt)
